# baseline (device time: 44552 ns/iter reference)
import jax
import jax.numpy as jnp
from jax import lax
from jax.experimental import pallas as pl
from jax.experimental.pallas import tpu as pltpu

N_DEV = 4
E_TOT = 16
E_LOC = E_TOT // N_DEV
CAP = 51
CAP_PAD = 64
BLK = E_LOC * CAP_PAD


def kernel(x, router_W, route_idx, expert_W):
    del router_W
    n_tok, d_model = x.shape
    h = expert_W.shape[2]

    my = lax.axis_index("i")

    e_idx = route_idx[:, 0]
    oh = e_idx[:, None] == jnp.arange(E_TOT, dtype=e_idx.dtype)
    ohi = oh.astype(jnp.int32)
    pos = jnp.cumsum(ohi, axis=0) - ohi
    keep = oh & (pos < CAP)
    slots = jnp.arange(CAP_PAD, dtype=jnp.int32)
    P = (keep.T[:, None, :] & (pos.T[:, None, :] == slots[None, :, None]))
    P = P.astype(jnp.bfloat16).reshape(N_DEV, BLK, n_tok)
    p_local = lax.dynamic_index_in_dim(P, my, axis=0, keepdims=False)
    order = (my - jnp.arange(N_DEV)) % N_DEV
    pt_rot = jnp.take(P, order, axis=0).reshape(N_DEV * BLK, n_tok).T
    pt_rot = pt_rot.astype(jnp.bfloat16)

    xb = x.astype(jnp.bfloat16)
    wb = expert_W.astype(jnp.bfloat16)

    def body(x_ref, w_ref, ploc_ref, pt_ref, out_ref, g_ref, xc_ref,
             send_sems, recv_sems):
        my_i = lax.axis_index("i")
        left = lax.rem(my_i - 1 + N_DEV, N_DEV)
        right = lax.rem(my_i + 1, N_DEV)

        barrier_sem = pltpu.get_barrier_semaphore()
        for nbr in (left, right):
            pl.semaphore_signal(
                barrier_sem, inc=1,
                device_id=(nbr,), device_id_type=pl.DeviceIdType.MESH,
            )
        pl.semaphore_wait(barrier_sem, 2)

        xc_ref[...] = jnp.dot(
            ploc_ref[...], x_ref[...], preferred_element_type=jnp.float32
        ).astype(jnp.bfloat16)
        for le in range(E_LOC):
            yc = jnp.dot(
                xc_ref[le * CAP_PAD:(le + 1) * CAP_PAD, :], w_ref[le],
                preferred_element_type=jnp.float32,
            )
            g_ref[0, le * CAP_PAD:(le + 1) * CAP_PAD, :] = yc.astype(
                jnp.bfloat16)

        for hop in range(N_DEV - 1):
            rdma = pltpu.make_async_remote_copy(
                src_ref=g_ref.at[hop],
                dst_ref=g_ref.at[hop + 1],
                send_sem=send_sems.at[hop],
                recv_sem=recv_sems.at[hop],
                device_id=(right,),
                device_id_type=pl.DeviceIdType.MESH,
            )
            rdma.start()
            rdma.wait()

        g = g_ref[...].reshape(N_DEV * BLK, h)
        out_ref[...] = jnp.dot(
            pt_ref[...], g, preferred_element_type=jnp.float32)

    return pl.pallas_call(
        body,
        out_shape=jax.ShapeDtypeStruct((n_tok, h), jnp.float32),
        in_specs=[
            pl.BlockSpec(memory_space=pltpu.VMEM),
            pl.BlockSpec(memory_space=pltpu.VMEM),
            pl.BlockSpec(memory_space=pltpu.VMEM),
            pl.BlockSpec(memory_space=pltpu.VMEM),
        ],
        out_specs=pl.BlockSpec(memory_space=pltpu.VMEM),
        scratch_shapes=[
            pltpu.VMEM((N_DEV, BLK, h), jnp.bfloat16),
            pltpu.VMEM((BLK, d_model), jnp.bfloat16),
            pltpu.SemaphoreType.DMA((N_DEV - 1,)),
            pltpu.SemaphoreType.DMA((N_DEV - 1,)),
        ],
        compiler_params=pltpu.CompilerParams(collective_id=0),
    )(xb, wb, p_local, pt_rot)


# device time: 42844 ns/iter; 1.0399x vs baseline; 1.0399x over previous
import jax
import jax.numpy as jnp
from jax import lax
from jax.experimental import pallas as pl
from jax.experimental.pallas import tpu as pltpu

N_DEV = 4
E_TOT = 16
E_LOC = E_TOT // N_DEV
CAP = 51
CAP_PAD = 64
BLK = E_LOC * CAP_PAD


def kernel(x, router_W, route_idx, expert_W):
    del router_W
    n_tok, d_model = x.shape
    h = expert_W.shape[2]

    my = lax.axis_index("i")

    e_idx = route_idx[:, 0].astype(jnp.int32)
    oh = e_idx[:, None] == jnp.arange(E_TOT, dtype=jnp.int32)
    ohi = oh.astype(jnp.int32)
    pos = jnp.cumsum(ohi, axis=0) - ohi
    pos_tok = jnp.take_along_axis(pos, e_idx[:, None], axis=1)[:, 0]
    kept = pos_tok < CAP
    blk = e_idx // E_LOC
    rel = (blk - my) % N_DEV
    slot_in_blk = (e_idx % E_LOC) * CAP_PAD + pos_tok
    tgt = jnp.where(kept, rel * BLK + slot_in_blk, -1)
    loc_tgt = jnp.where(kept & (blk == my), slot_in_blk, -1)

    xb = x.astype(jnp.bfloat16)
    wb = expert_W.astype(jnp.bfloat16)

    def body(x_ref, w_ref, loc_ref, tgt_ref, out_ref, g_ref,
             send_sems, recv_sems):
        my_i = lax.axis_index("i")
        left = lax.rem(my_i - 1 + N_DEV, N_DEV)
        right = lax.rem(my_i + 1, N_DEV)

        barrier_sem = pltpu.get_barrier_semaphore()
        for nbr in (left, right):
            pl.semaphore_signal(
                barrier_sem, inc=1,
                device_id=(nbr,), device_id_type=pl.DeviceIdType.MESH,
            )
        pl.semaphore_wait(barrier_sem, 2)

        row_iota = lax.broadcasted_iota(jnp.int32, (BLK, n_tok), 0)
        p_local = (loc_ref[...] == row_iota).astype(jnp.bfloat16)
        xc = jnp.dot(
            p_local, x_ref[...], preferred_element_type=jnp.float32
        ).astype(jnp.bfloat16)
        for le in range(E_LOC):
            yc = jnp.dot(
                xc[le * CAP_PAD:(le + 1) * CAP_PAD, :], w_ref[le],
                preferred_element_type=jnp.float32,
            )
            g_ref[0, le * CAP_PAD:(le + 1) * CAP_PAD, :] = yc.astype(
                jnp.bfloat16)

        send_l1 = pltpu.make_async_remote_copy(
            src_ref=g_ref.at[0], dst_ref=g_ref.at[1],
            send_sem=send_sems.at[0], recv_sem=recv_sems.at[0],
            device_id=(left,), device_id_type=pl.DeviceIdType.MESH,
        )
        send_r1 = pltpu.make_async_remote_copy(
            src_ref=g_ref.at[0], dst_ref=g_ref.at[3],
            send_sem=send_sems.at[1], recv_sem=recv_sems.at[1],
            device_id=(right,), device_id_type=pl.DeviceIdType.MESH,
        )
        send_l1.start()
        send_r1.start()

        col_iota = lax.broadcasted_iota(jnp.int32, (n_tok, N_DEV * BLK), 1)
        pt = (tgt_ref[...] == col_iota).astype(jnp.bfloat16)
        out_ref[...] = jnp.dot(
            pt[:, 0:BLK], g_ref[0], preferred_element_type=jnp.float32)

        send_l1.wait_recv()
        send_l2 = pltpu.make_async_remote_copy(
            src_ref=g_ref.at[1], dst_ref=g_ref.at[2],
            send_sem=send_sems.at[2], recv_sem=recv_sems.at[2],
            device_id=(left,), device_id_type=pl.DeviceIdType.MESH,
        )
        send_l2.start()
        out_ref[...] += jnp.dot(
            pt[:, BLK:2 * BLK], g_ref[1], preferred_element_type=jnp.float32)

        send_r1.wait_recv()
        out_ref[...] += jnp.dot(
            pt[:, 3 * BLK:4 * BLK], g_ref[3],
            preferred_element_type=jnp.float32)

        send_l2.wait_recv()
        out_ref[...] += jnp.dot(
            pt[:, 2 * BLK:3 * BLK], g_ref[2],
            preferred_element_type=jnp.float32)

        send_l1.wait_send()
        send_r1.wait_send()
        send_l2.wait_send()

    return pl.pallas_call(
        body,
        out_shape=jax.ShapeDtypeStruct((n_tok, h), jnp.float32),
        in_specs=[
            pl.BlockSpec(memory_space=pltpu.VMEM),
            pl.BlockSpec(memory_space=pltpu.VMEM),
            pl.BlockSpec(memory_space=pltpu.VMEM),
            pl.BlockSpec(memory_space=pltpu.VMEM),
        ],
        out_specs=pl.BlockSpec(memory_space=pltpu.VMEM),
        scratch_shapes=[
            pltpu.VMEM((N_DEV, BLK, h), jnp.bfloat16),
            pltpu.SemaphoreType.DMA((3,)),
            pltpu.SemaphoreType.DMA((3,)),
        ],
        compiler_params=pltpu.CompilerParams(collective_id=0),
    )(xb, wb, loc_tgt[None, :], tgt[:, None])


# device time: 29105 ns/iter; 1.5307x vs baseline; 1.4720x over previous
import jax
import jax.numpy as jnp
from jax import lax
from jax.experimental import pallas as pl
from jax.experimental.pallas import tpu as pltpu

N_DEV = 4
E_TOT = 16
E_LOC = E_TOT // N_DEV
CAP = 51
CAP_PAD = 64
BLK = E_LOC * CAP_PAD


def kernel(x, router_W, route_idx, expert_W):
    del router_W
    n_tok, d_model = x.shape
    h = expert_W.shape[2]

    xb = x.astype(jnp.bfloat16)
    wb = expert_W.astype(jnp.bfloat16)
    e_row = route_idx.astype(jnp.int32)
    e_col = e_row.T

    def body(x_ref, w_ref, er_ref, ec_ref, out_ref, g_ref,
             send_sems, recv_sems):
        my_i = lax.axis_index("i")
        left = lax.rem(my_i - 1 + N_DEV, N_DEV)
        right = lax.rem(my_i + 1, N_DEV)

        barrier_sem = pltpu.get_barrier_semaphore()
        for nbr in (left, right):
            pl.semaphore_signal(
                barrier_sem, inc=1,
                device_id=(nbr,), device_id_type=pl.DeviceIdType.MESH,
            )

        er = er_ref[...]
        ec = ec_ref[...]
        ri = lax.broadcasted_iota(jnp.int32, (n_tok, n_tok), 0)
        ci = lax.broadcasted_iota(jnp.int32, (n_tok, n_tok), 1)
        same = er == ec
        pos_col = jnp.sum(((ri < ci) & same).astype(jnp.int32),
                          axis=0, keepdims=True)

        kept_c = pos_col < CAP
        loc_c = lax.div(ec, E_LOC) == my_i
        loc_tgt = jnp.where(kept_c & loc_c,
                            lax.rem(ec, E_LOC) * CAP_PAD + pos_col, -1)
        r_iota = lax.broadcasted_iota(jnp.int32, (BLK, n_tok), 0)
        p_local = (loc_tgt == r_iota).astype(jnp.bfloat16)

        xc = jnp.dot(
            p_local, x_ref[...], preferred_element_type=jnp.float32
        ).astype(jnp.bfloat16)
        for le in range(E_LOC):
            yc = jnp.dot(
                xc[le * CAP_PAD:(le + 1) * CAP_PAD, :], w_ref[le],
                preferred_element_type=jnp.float32,
            )
            g_ref[0, le * CAP_PAD:(le + 1) * CAP_PAD, :] = yc.astype(
                jnp.bfloat16)

        pl.semaphore_wait(barrier_sem, 2)

        send_l1 = pltpu.make_async_remote_copy(
            src_ref=g_ref.at[0], dst_ref=g_ref.at[1],
            send_sem=send_sems.at[0], recv_sem=recv_sems.at[0],
            device_id=(left,), device_id_type=pl.DeviceIdType.MESH,
        )
        send_r1 = pltpu.make_async_remote_copy(
            src_ref=g_ref.at[0], dst_ref=g_ref.at[3],
            send_sem=send_sems.at[1], recv_sem=recv_sems.at[1],
            device_id=(right,), device_id_type=pl.DeviceIdType.MESH,
        )
        send_l1.start()
        send_r1.start()

        pos_row = jnp.sum(((ri > ci) & same).astype(jnp.int32),
                          axis=1, keepdims=True)
        kept_r = pos_row < CAP
        rel_r = lax.rem(lax.div(er, E_LOC) - my_i + N_DEV, N_DEV)
        tgt = jnp.where(kept_r,
                        rel_r * BLK + lax.rem(er, E_LOC) * CAP_PAD + pos_row,
                        -1)
        c_iota = lax.broadcasted_iota(jnp.int32, (n_tok, N_DEV * BLK), 1)
        pt = (tgt == c_iota).astype(jnp.bfloat16)

        send_l1.wait_recv()
        send_l2 = pltpu.make_async_remote_copy(
            src_ref=g_ref.at[1], dst_ref=g_ref.at[2],
            send_sem=send_sems.at[2], recv_sem=recv_sems.at[2],
            device_id=(left,), device_id_type=pl.DeviceIdType.MESH,
        )
        send_l2.start()

        out_ref[...] = jnp.dot(
            pt[:, :2 * BLK], g_ref[0:2].reshape(2 * BLK, h),
            preferred_element_type=jnp.float32)

        send_r1.wait_recv()
        send_l2.wait_recv()
        out_ref[...] += jnp.dot(
            pt[:, 2 * BLK:], g_ref[2:4].reshape(2 * BLK, h),
            preferred_element_type=jnp.float32)

        send_l1.wait_send()
        send_r1.wait_send()
        send_l2.wait_send()

    return pl.pallas_call(
        body,
        out_shape=jax.ShapeDtypeStruct((n_tok, h), jnp.float32),
        in_specs=[
            pl.BlockSpec(memory_space=pltpu.VMEM),
            pl.BlockSpec(memory_space=pltpu.VMEM),
            pl.BlockSpec(memory_space=pltpu.VMEM),
            pl.BlockSpec(memory_space=pltpu.VMEM),
        ],
        out_specs=pl.BlockSpec(memory_space=pltpu.VMEM),
        scratch_shapes=[
            pltpu.VMEM((N_DEV, BLK, h), jnp.bfloat16),
            pltpu.SemaphoreType.DMA((3,)),
            pltpu.SemaphoreType.DMA((3,)),
        ],
        compiler_params=pltpu.CompilerParams(collective_id=0),
    )(xb, wb, e_row, e_col)


# device time: 26568 ns/iter; 1.6769x vs baseline; 1.0955x over previous
import jax
import jax.numpy as jnp
from jax import lax
from jax.experimental import pallas as pl
from jax.experimental.pallas import tpu as pltpu

N_DEV = 4
E_TOT = 16
E_LOC = E_TOT // N_DEV
CAP = 51
CAP_PAD = 64
BLK = E_LOC * CAP_PAD
HALF = BLK // 2


def kernel(x, router_W, route_idx, expert_W):
    del router_W
    n_tok, d_model = x.shape
    h = expert_W.shape[2]

    e_row = route_idx.astype(jnp.int32)
    e_col = e_row.T

    def body(x_ref, w_ref, er_ref, ec_ref, out_ref, g_ref,
             send_sems, recv_sems):
        my_i = lax.axis_index("i")
        left = lax.rem(my_i - 1 + N_DEV, N_DEV)
        right = lax.rem(my_i + 1, N_DEV)

        barrier_sem = pltpu.get_barrier_semaphore()
        for nbr in (left, right):
            pl.semaphore_signal(
                barrier_sem, inc=1,
                device_id=(nbr,), device_id_type=pl.DeviceIdType.MESH,
            )

        er = er_ref[...]
        ec = ec_ref[...]
        ri = lax.broadcasted_iota(jnp.int32, (n_tok, n_tok), 0)
        ci = lax.broadcasted_iota(jnp.int32, (n_tok, n_tok), 1)
        same = er == ec
        pos_col = jnp.sum(((ri < ci) & same).astype(jnp.int32),
                          axis=0, keepdims=True)

        kept_c = pos_col < CAP
        loc_c = lax.div(ec, E_LOC) == my_i
        loc_tgt = jnp.where(kept_c & loc_c,
                            lax.rem(ec, E_LOC) * CAP_PAD + pos_col, -1)
        r_iota = lax.broadcasted_iota(jnp.int32, (BLK, n_tok), 0)
        p_local = (loc_tgt == r_iota).astype(jnp.bfloat16)

        xc = jnp.dot(
            p_local, x_ref[...].astype(jnp.bfloat16),
            preferred_element_type=jnp.float32,
        ).astype(jnp.bfloat16)
        for le in range(E_LOC):
            yc = jnp.dot(
                xc[le * CAP_PAD:(le + 1) * CAP_PAD, :],
                w_ref[le].astype(jnp.bfloat16),
                preferred_element_type=jnp.float32,
            )
            g_ref[0, le * CAP_PAD:(le + 1) * CAP_PAD, :] = yc.astype(
                jnp.bfloat16)

        pl.semaphore_wait(barrier_sem, 2)

        send_l1 = pltpu.make_async_remote_copy(
            src_ref=g_ref.at[0], dst_ref=g_ref.at[1],
            send_sem=send_sems.at[0], recv_sem=recv_sems.at[0],
            device_id=(left,), device_id_type=pl.DeviceIdType.MESH,
        )
        send_r1 = pltpu.make_async_remote_copy(
            src_ref=g_ref.at[0], dst_ref=g_ref.at[3],
            send_sem=send_sems.at[1], recv_sem=recv_sems.at[1],
            device_id=(right,), device_id_type=pl.DeviceIdType.MESH,
        )
        send_l1.start()
        send_r1.start()

        pos_row = jnp.sum(((ri > ci) & same).astype(jnp.int32),
                          axis=1, keepdims=True)
        kept_r = pos_row < CAP
        rel_r = lax.rem(lax.div(er, E_LOC) - my_i + N_DEV, N_DEV)
        tgt = jnp.where(kept_r,
                        rel_r * BLK + lax.rem(er, E_LOC) * CAP_PAD + pos_row,
                        -1)
        c_iota = lax.broadcasted_iota(jnp.int32, (n_tok, N_DEV * BLK), 1)
        pt = (tgt == c_iota).astype(jnp.bfloat16)

        send_l1.wait_recv()
        send_r1.wait_recv()
        send_l2 = pltpu.make_async_remote_copy(
            src_ref=g_ref.at[1, pl.ds(0, HALF)],
            dst_ref=g_ref.at[2, pl.ds(0, HALF)],
            send_sem=send_sems.at[2], recv_sem=recv_sems.at[2],
            device_id=(left,), device_id_type=pl.DeviceIdType.MESH,
        )
        send_r2 = pltpu.make_async_remote_copy(
            src_ref=g_ref.at[3, pl.ds(HALF, HALF)],
            dst_ref=g_ref.at[2, pl.ds(HALF, HALF)],
            send_sem=send_sems.at[3], recv_sem=recv_sems.at[3],
            device_id=(right,), device_id_type=pl.DeviceIdType.MESH,
        )
        send_l2.start()
        send_r2.start()

        out_ref[...] = jnp.dot(
            pt[:, :2 * BLK], g_ref[0:2].reshape(2 * BLK, h),
            preferred_element_type=jnp.float32)

        send_l2.wait_recv()
        send_r2.wait_recv()
        out_ref[...] += jnp.dot(
            pt[:, 2 * BLK:], g_ref[2:4].reshape(2 * BLK, h),
            preferred_element_type=jnp.float32)

        send_l1.wait_send()
        send_r1.wait_send()
        send_l2.wait_send()
        send_r2.wait_send()

    return pl.pallas_call(
        body,
        out_shape=jax.ShapeDtypeStruct((n_tok, h), jnp.float32),
        in_specs=[
            pl.BlockSpec(memory_space=pltpu.VMEM),
            pl.BlockSpec(memory_space=pltpu.VMEM),
            pl.BlockSpec(memory_space=pltpu.VMEM),
            pl.BlockSpec(memory_space=pltpu.VMEM),
        ],
        out_specs=pl.BlockSpec(memory_space=pltpu.VMEM),
        scratch_shapes=[
            pltpu.VMEM((N_DEV, BLK, h), jnp.bfloat16),
            pltpu.SemaphoreType.DMA((4,)),
            pltpu.SemaphoreType.DMA((4,)),
        ],
        compiler_params=pltpu.CompilerParams(collective_id=0),
    )(x, expert_W, e_row, e_col)


# device time: 25411 ns/iter; 1.7533x vs baseline; 1.0455x over previous
import jax
import jax.numpy as jnp
from jax import lax
from jax.experimental import pallas as pl
from jax.experimental.pallas import tpu as pltpu

N_DEV = 4
E_TOT = 16
E_LOC = E_TOT // N_DEV
CAP = 51
CAP_PAD = 64
BLK = E_LOC * CAP_PAD
HALF = BLK // 2


def kernel(x, router_W, route_idx, expert_W):
    del router_W
    n_tok, d_model = x.shape
    h = expert_W.shape[2]

    e_row = route_idx.astype(jnp.int32)
    e_col = e_row.T

    def body(x_ref, w_ref, er_ref, ec_ref, out_ref, g_ref,
             send_sems, recv_sems):
        my_i = lax.axis_index("i")
        left = lax.rem(my_i - 1 + N_DEV, N_DEV)
        right = lax.rem(my_i + 1, N_DEV)

        barrier_sem = pltpu.get_barrier_semaphore()
        for nbr in (left, right):
            pl.semaphore_signal(
                barrier_sem, inc=1,
                device_id=(nbr,), device_id_type=pl.DeviceIdType.MESH,
            )

        er = er_ref[...]
        ec = ec_ref[...]
        ri = lax.broadcasted_iota(jnp.int32, (n_tok, n_tok), 0)
        ci = lax.broadcasted_iota(jnp.int32, (n_tok, n_tok), 1)
        same = er == ec
        pos_col = jnp.sum(((ri < ci) & same).astype(jnp.int32),
                          axis=0, keepdims=True)

        kept_c = pos_col < CAP
        loc_c = lax.div(ec, E_LOC) == my_i
        loc_tgt = jnp.where(kept_c & loc_c,
                            lax.rem(ec, E_LOC) * CAP_PAD + pos_col, -1)
        r_iota = lax.broadcasted_iota(jnp.int32, (BLK, n_tok), 0)
        p_local = (loc_tgt == r_iota).astype(jnp.bfloat16)

        xc = jnp.dot(
            p_local, x_ref[...].astype(jnp.bfloat16),
            preferred_element_type=jnp.float32,
        ).astype(jnp.bfloat16)
        for le in range(2):
            yc = jnp.dot(
                xc[le * CAP_PAD:(le + 1) * CAP_PAD, :],
                w_ref[le].astype(jnp.bfloat16),
                preferred_element_type=jnp.float32,
            )
            g_ref[0, le * CAP_PAD:(le + 1) * CAP_PAD, :] = yc.astype(
                jnp.bfloat16)

        pl.semaphore_wait(barrier_sem, 2)
        s0 = pltpu.make_async_remote_copy(
            src_ref=g_ref.at[0, pl.ds(0, HALF)],
            dst_ref=g_ref.at[1, pl.ds(0, HALF)],
            send_sem=send_sems.at[0], recv_sem=recv_sems.at[0],
            device_id=(left,), device_id_type=pl.DeviceIdType.MESH,
        )
        s1 = pltpu.make_async_remote_copy(
            src_ref=g_ref.at[0, pl.ds(0, HALF)],
            dst_ref=g_ref.at[3, pl.ds(0, HALF)],
            send_sem=send_sems.at[1], recv_sem=recv_sems.at[1],
            device_id=(right,), device_id_type=pl.DeviceIdType.MESH,
        )
        s0.start()
        s1.start()

        for le in range(2, E_LOC):
            yc = jnp.dot(
                xc[le * CAP_PAD:(le + 1) * CAP_PAD, :],
                w_ref[le].astype(jnp.bfloat16),
                preferred_element_type=jnp.float32,
            )
            g_ref[0, le * CAP_PAD:(le + 1) * CAP_PAD, :] = yc.astype(
                jnp.bfloat16)

        s2 = pltpu.make_async_remote_copy(
            src_ref=g_ref.at[0, pl.ds(HALF, HALF)],
            dst_ref=g_ref.at[1, pl.ds(HALF, HALF)],
            send_sem=send_sems.at[2], recv_sem=recv_sems.at[2],
            device_id=(left,), device_id_type=pl.DeviceIdType.MESH,
        )
        s3 = pltpu.make_async_remote_copy(
            src_ref=g_ref.at[0, pl.ds(HALF, HALF)],
            dst_ref=g_ref.at[3, pl.ds(HALF, HALF)],
            send_sem=send_sems.at[3], recv_sem=recv_sems.at[3],
            device_id=(right,), device_id_type=pl.DeviceIdType.MESH,
        )
        s2.start()
        s3.start()

        pos_row = jnp.sum(((ri > ci) & same).astype(jnp.int32),
                          axis=1, keepdims=True)
        kept_r = pos_row < CAP
        rel_r = lax.rem(lax.div(er, E_LOC) - my_i + N_DEV, N_DEV)
        tgt = jnp.where(kept_r,
                        rel_r * BLK + lax.rem(er, E_LOC) * CAP_PAD + pos_row,
                        -1)
        c_iota = lax.broadcasted_iota(jnp.int32, (n_tok, N_DEV * BLK), 1)
        pt = (tgt == c_iota).astype(jnp.bfloat16)

        s0.wait_recv()
        s4 = pltpu.make_async_remote_copy(
            src_ref=g_ref.at[1, pl.ds(0, HALF)],
            dst_ref=g_ref.at[2, pl.ds(0, HALF)],
            send_sem=send_sems.at[4], recv_sem=recv_sems.at[4],
            device_id=(left,), device_id_type=pl.DeviceIdType.MESH,
        )
        s4.start()
        s3.wait_recv()
        s5 = pltpu.make_async_remote_copy(
            src_ref=g_ref.at[3, pl.ds(HALF, HALF)],
            dst_ref=g_ref.at[2, pl.ds(HALF, HALF)],
            send_sem=send_sems.at[5], recv_sem=recv_sems.at[5],
            device_id=(right,), device_id_type=pl.DeviceIdType.MESH,
        )
        s5.start()

        s2.wait_recv()
        out_ref[...] = jnp.dot(
            pt[:, :2 * BLK], g_ref[0:2].reshape(2 * BLK, h),
            preferred_element_type=jnp.float32)

        s1.wait_recv()
        s4.wait_recv()
        s5.wait_recv()
        out_ref[...] += jnp.dot(
            pt[:, 2 * BLK:], g_ref[2:4].reshape(2 * BLK, h),
            preferred_element_type=jnp.float32)

        for s in (s0, s1, s2, s3, s4, s5):
            s.wait_send()

    return pl.pallas_call(
        body,
        out_shape=jax.ShapeDtypeStruct((n_tok, h), jnp.float32),
        in_specs=[
            pl.BlockSpec(memory_space=pltpu.VMEM),
            pl.BlockSpec(memory_space=pltpu.VMEM),
            pl.BlockSpec(memory_space=pltpu.VMEM),
            pl.BlockSpec(memory_space=pltpu.VMEM),
        ],
        out_specs=pl.BlockSpec(memory_space=pltpu.VMEM),
        scratch_shapes=[
            pltpu.VMEM((N_DEV, BLK, h), jnp.bfloat16),
            pltpu.SemaphoreType.DMA((6,)),
            pltpu.SemaphoreType.DMA((6,)),
        ],
        compiler_params=pltpu.CompilerParams(collective_id=0),
    )(x, expert_W, e_row, e_col)


# device time: 23727 ns/iter; 1.8777x vs baseline; 1.0710x over previous
import jax
import jax.numpy as jnp
from jax import lax
from jax.experimental import pallas as pl
from jax.experimental.pallas import tpu as pltpu

N_DEV = 4
E_TOT = 16
E_LOC = E_TOT // N_DEV
CAP = 51
CAP_PAD = 56
BLK = E_LOC * CAP_PAD
HALF = BLK // 2


def kernel(x, router_W, route_idx, expert_W):
    del router_W
    n_tok, d_model = x.shape
    h = expert_W.shape[2]

    e_row = route_idx.astype(jnp.int32)
    e_col = e_row.T

    def body(x_ref, w_ref, er_ref, ec_ref, out_ref, g_ref,
             send_sems, recv_sems):
        my_i = lax.axis_index("i")
        left = lax.rem(my_i - 1 + N_DEV, N_DEV)
        right = lax.rem(my_i + 1, N_DEV)

        barrier_sem = pltpu.get_barrier_semaphore()
        for nbr in (left, right):
            pl.semaphore_signal(
                barrier_sem, inc=1,
                device_id=(nbr,), device_id_type=pl.DeviceIdType.MESH,
            )

        er = er_ref[...]
        ec = ec_ref[...]
        ri = lax.broadcasted_iota(jnp.int32, (n_tok, n_tok), 0)
        ci = lax.broadcasted_iota(jnp.int32, (n_tok, n_tok), 1)
        same = er == ec
        pos_col = jnp.sum(((ri < ci) & same).astype(jnp.int32),
                          axis=0, keepdims=True)

        kept_c = pos_col < CAP
        loc_c = lax.div(ec, E_LOC) == my_i
        loc_tgt = jnp.where(kept_c & loc_c,
                            lax.rem(ec, E_LOC) * CAP_PAD + pos_col, -1)
        r_iota = lax.broadcasted_iota(jnp.int32, (BLK, n_tok), 0)
        p_local = (loc_tgt == r_iota).astype(jnp.bfloat16)

        xc = jnp.dot(
            p_local, x_ref[...].astype(jnp.bfloat16),
            preferred_element_type=jnp.float32,
        ).astype(jnp.bfloat16)
        for le in range(2):
            yc = jnp.dot(
                xc[le * CAP_PAD:(le + 1) * CAP_PAD, :],
                w_ref[le].astype(jnp.bfloat16),
                preferred_element_type=jnp.float32,
            )
            g_ref[0, le * CAP_PAD:(le + 1) * CAP_PAD, :] = yc.astype(
                jnp.bfloat16)

        pl.semaphore_wait(barrier_sem, 2)
        s0 = pltpu.make_async_remote_copy(
            src_ref=g_ref.at[0, pl.ds(0, HALF)],
            dst_ref=g_ref.at[1, pl.ds(0, HALF)],
            send_sem=send_sems.at[0], recv_sem=recv_sems.at[0],
            device_id=(left,), device_id_type=pl.DeviceIdType.MESH,
        )
        s1 = pltpu.make_async_remote_copy(
            src_ref=g_ref.at[0, pl.ds(0, HALF)],
            dst_ref=g_ref.at[3, pl.ds(0, HALF)],
            send_sem=send_sems.at[1], recv_sem=recv_sems.at[1],
            device_id=(right,), device_id_type=pl.DeviceIdType.MESH,
        )
        s0.start()
        s1.start()

        for le in range(2, E_LOC):
            yc = jnp.dot(
                xc[le * CAP_PAD:(le + 1) * CAP_PAD, :],
                w_ref[le].astype(jnp.bfloat16),
                preferred_element_type=jnp.float32,
            )
            g_ref[0, le * CAP_PAD:(le + 1) * CAP_PAD, :] = yc.astype(
                jnp.bfloat16)

        s2 = pltpu.make_async_remote_copy(
            src_ref=g_ref.at[0, pl.ds(HALF, HALF)],
            dst_ref=g_ref.at[1, pl.ds(HALF, HALF)],
            send_sem=send_sems.at[2], recv_sem=recv_sems.at[2],
            device_id=(left,), device_id_type=pl.DeviceIdType.MESH,
        )
        s3 = pltpu.make_async_remote_copy(
            src_ref=g_ref.at[0, pl.ds(HALF, HALF)],
            dst_ref=g_ref.at[3, pl.ds(HALF, HALF)],
            send_sem=send_sems.at[3], recv_sem=recv_sems.at[3],
            device_id=(right,), device_id_type=pl.DeviceIdType.MESH,
        )
        s2.start()
        s3.start()

        pos_row = jnp.sum(((ri > ci) & same).astype(jnp.int32),
                          axis=1, keepdims=True)
        kept_r = pos_row < CAP
        rel_r = lax.rem(lax.div(er, E_LOC) - my_i + N_DEV, N_DEV)
        tgt = jnp.where(kept_r,
                        rel_r * BLK + lax.rem(er, E_LOC) * CAP_PAD + pos_row,
                        -1)
        c_iota = lax.broadcasted_iota(jnp.int32, (n_tok, N_DEV * BLK), 1)
        pt = (tgt == c_iota).astype(jnp.bfloat16)

        s0.wait_recv()
        s4 = pltpu.make_async_remote_copy(
            src_ref=g_ref.at[1, pl.ds(0, HALF)],
            dst_ref=g_ref.at[2, pl.ds(0, HALF)],
            send_sem=send_sems.at[4], recv_sem=recv_sems.at[4],
            device_id=(left,), device_id_type=pl.DeviceIdType.MESH,
        )
        s4.start()
        s3.wait_recv()
        s5 = pltpu.make_async_remote_copy(
            src_ref=g_ref.at[3, pl.ds(HALF, HALF)],
            dst_ref=g_ref.at[2, pl.ds(HALF, HALF)],
            send_sem=send_sems.at[5], recv_sem=recv_sems.at[5],
            device_id=(right,), device_id_type=pl.DeviceIdType.MESH,
        )
        s5.start()

        s2.wait_recv()
        out_ref[...] = jnp.dot(
            pt[:, :2 * BLK], g_ref[0:2].reshape(2 * BLK, h),
            preferred_element_type=jnp.float32).astype(jnp.bfloat16)

        s1.wait_recv()
        s4.wait_recv()
        s5.wait_recv()
        out_ref[...] = (out_ref[...] + jnp.dot(
            pt[:, 2 * BLK:], g_ref[2:4].reshape(2 * BLK, h),
            preferred_element_type=jnp.float32).astype(jnp.bfloat16))

        for s in (s0, s1, s2, s3, s4, s5):
            s.wait_send()

    return pl.pallas_call(
        body,
        out_shape=jax.ShapeDtypeStruct((n_tok, h), jnp.bfloat16),
        in_specs=[
            pl.BlockSpec(memory_space=pltpu.VMEM),
            pl.BlockSpec(memory_space=pltpu.VMEM),
            pl.BlockSpec(memory_space=pltpu.VMEM),
            pl.BlockSpec(memory_space=pltpu.VMEM),
        ],
        out_specs=pl.BlockSpec(memory_space=pltpu.VMEM),
        scratch_shapes=[
            pltpu.VMEM((N_DEV, BLK, h), jnp.bfloat16),
            pltpu.SemaphoreType.DMA((6,)),
            pltpu.SemaphoreType.DMA((6,)),
        ],
        compiler_params=pltpu.CompilerParams(collective_id=0),
    )(x, expert_W, e_row, e_col)
